# baseline (speedup 1.0000x reference)
import functools

import jax
import jax.numpy as jnp
from jax import lax
from jax.experimental import pallas as pl
from jax.experimental.pallas import tpu as pltpu
from jax.experimental.pallas import tpu_sc as plsc

_B, _L = 4096, 200
_N = _B * _L
_DW, _DP = 128, 16
_D = _DW + _DP
_NC, _NS = 2, 16
_NW = _NC * _NS
_RW = _B // _NW
_PW = _RW * _L
_GA, _GB = 128, _L - 128
_PP = _PW // 2
_CH = 1600


def _build():
  mesh = plsc.VectorSubcoreMesh(core_axis_name="c", subcore_axis_name="s")

  @functools.partial(
      pl.kernel,
      mesh=mesh,
      compiler_params=pltpu.CompilerParams(
          use_tc_tiling_on_sc=False, needs_layout_passes=False),
      out_type=jax.ShapeDtypeStruct((_B, _L, _D), jnp.float32),
      scratch_types=[
          pltpu.VMEM((_CH,), jnp.int32),
          pltpu.VMEM((_PW,), jnp.int32),
          pltpu.VMEM((_PW,), jnp.int32),
          pltpu.VMEM((2, _L, _DW), jnp.float32),
          pltpu.VMEM((2, _L, _DP), jnp.float32),
          pltpu.SemaphoreType.DMA,
          pltpu.SemaphoreType.DMA,
          pltpu.SemaphoreType.DMA,
          pltpu.SemaphoreType.DMA,
      ],
  )
  def emb(packed_hbm, ww_hbm, wp_hbm, out_hbm,
          ci, xi, pi, wrows, prows, gs0, gs1, ws0, ws1):
    gsem = (gs0, gs1)
    wsem = (ws0, ws1)
    wid = lax.axis_index("s") * _NC + lax.axis_index("c")
    rowbase = wid * _RW
    pbase = wid * _PP

    lanes = lax.iota(jnp.int32, 16)

    @pl.loop(0, _PP, step=_CH)
    def _unpack(off):
      pltpu.sync_copy(packed_hbm.at[pl.ds(pbase + off, _CH)], ci)

      @pl.loop(0, _CH, step=16, unroll=8)
      def _vec(j):
        v = ci[pl.ds(j, 16)]
        lo = lax.bitwise_and(v, 0xFFFF)
        hi = lax.shift_right_logical(v, 16)
        t = (off + j) * 2 + lanes * 2
        plsc.store_scatter(xi, [t], lax.shift_right_logical(lo, 5))
        plsc.store_scatter(pi, [t], lax.bitwise_and(lo, 31))
        plsc.store_scatter(xi, [t + 1], lax.shift_right_logical(hi, 5))
        plsc.store_scatter(pi, [t + 1], lax.bitwise_and(hi, 31))

    def issue_gather(r, b):
      sa = pl.ds(r * _L, _GA)
      sb = pl.ds(r * _L + _GA, _GB)
      pltpu.async_copy(ww_hbm.at[xi.at[sa]], wrows.at[b, pl.ds(0, _GA)],
                       gsem[b])
      pltpu.async_copy(ww_hbm.at[xi.at[sb]], wrows.at[b, pl.ds(_GA, _GB)],
                       gsem[b])
      pltpu.async_copy(wp_hbm.at[pi.at[sa]], prows.at[b, pl.ds(0, _GA)],
                       gsem[b])
      pltpu.async_copy(wp_hbm.at[pi.at[sb]], prows.at[b, pl.ds(_GA, _GB)],
                       gsem[b])

    def wait_gather(b):
      pltpu.make_async_copy(
          ww_hbm.at[xi.at[pl.ds(0, _GA)]], wrows.at[b, pl.ds(0, _GA)],
          gsem[b]).wait()
      pltpu.make_async_copy(
          ww_hbm.at[xi.at[pl.ds(0, _GB)]], wrows.at[b, pl.ds(_GA, _GB)],
          gsem[b]).wait()
      pltpu.make_async_copy(
          wp_hbm.at[pi.at[pl.ds(0, _GA)]], prows.at[b, pl.ds(0, _GA)],
          gsem[b]).wait()
      pltpu.make_async_copy(
          wp_hbm.at[pi.at[pl.ds(0, _GB)]], prows.at[b, pl.ds(_GA, _GB)],
          gsem[b]).wait()

    def issue_write(r, b):
      row = rowbase + r
      pltpu.async_copy(
          wrows.at[b], out_hbm.at[row, pl.ds(0, _L), pl.ds(0, _DW)], wsem[b])
      pltpu.async_copy(
          prows.at[b], out_hbm.at[row, pl.ds(0, _L), pl.ds(_DW, _DP)],
          wsem[b])

    def wait_write(b):
      pltpu.make_async_copy(
          wrows.at[b], out_hbm.at[0, pl.ds(0, _L), pl.ds(0, _DW)],
          wsem[b]).wait()
      pltpu.make_async_copy(
          prows.at[b], out_hbm.at[0, pl.ds(0, _L), pl.ds(_DW, _DP)],
          wsem[b]).wait()

    issue_gather(0, 0)
    issue_gather(1, 1)

    @pl.loop(0, _RW, step=2)
    def _rows(r0):
      for b in range(2):
        r = r0 + b
        wait_gather(b)
        issue_write(r, b)
        wait_write(b)

        @pl.when(r + 2 < _RW)
        def _():
          issue_gather(r + 2, b)

  return emb


_emb = _build()


@jax.jit
def kernel(x, pos, W_word, W_pos):
  code = x.astype(jnp.int32) * 32 + pos.astype(jnp.int32)
  packed = (code[:, 0::2] | (code[:, 1::2] << 16)).reshape(_N // 2)
  return _emb(packed, W_word, W_pos)


# speedup vs baseline: 2.2830x; 2.2830x over previous
import functools

import jax
import jax.numpy as jnp
from jax import lax
from jax.experimental import pallas as pl
from jax.experimental.pallas import tpu as pltpu
from jax.experimental.pallas import tpu_sc as plsc

_B, _L = 4096, 200
_N = _B * _L
_DW, _DP = 128, 16
_D = _DW + _DP
_NC, _NS = 2, 16
_NW = _NC * _NS
_RW = _B // _NW
_CR = 32
_NCH = _RW // _CR
_GA, _GB = 128, _L - 128


def _build():
  mesh = plsc.VectorSubcoreMesh(core_axis_name="c", subcore_axis_name="s")

  @functools.partial(
      pl.kernel,
      mesh=mesh,
      compiler_params=pltpu.CompilerParams(needs_layout_passes=False),
      out_type=jax.ShapeDtypeStruct((_B, _L, _D), jnp.float32),
      scratch_types=[
          pltpu.VMEM((_CR, _L), jnp.int32),
          pltpu.VMEM((_CR, _L), jnp.int32),
          pltpu.VMEM((24, _DP), jnp.float32),
          pltpu.VMEM((2, _L, _DW), jnp.float32),
          pltpu.VMEM((2, _L, _DP), jnp.float32),
          pltpu.SemaphoreType.DMA,
          pltpu.SemaphoreType.DMA,
          pltpu.SemaphoreType.DMA,
          pltpu.SemaphoreType.DMA,
      ],
  )
  def emb(x_hbm, p_hbm, ww_hbm, wp_hbm, out_hbm,
          xi, pi, wp_v, wrows, prows, gs0, gs1, ws0, ws1):
    gsem = (gs0, gs1)
    wsem = (ws0, ws1)
    wid = lax.axis_index("s") * _NC + lax.axis_index("c")
    rowbase = wid * _RW
    lanes = lax.iota(jnp.int32, 16)
    pltpu.sync_copy(wp_hbm, wp_v)

    def issue_gather(rl, b):
      pltpu.async_copy(ww_hbm.at[xi.at[rl, pl.ds(0, _GA)]],
                       wrows.at[b, pl.ds(0, _GA)], gsem[b])
      pltpu.async_copy(ww_hbm.at[xi.at[rl, pl.ds(_GA, _GB)]],
                       wrows.at[b, pl.ds(_GA, _GB)], gsem[b])

    def wait_gather(b):
      pltpu.make_async_copy(
          ww_hbm.at[xi.at[0, pl.ds(0, _GA)]], wrows.at[b, pl.ds(0, _GA)],
          gsem[b]).wait()
      pltpu.make_async_copy(
          ww_hbm.at[xi.at[0, pl.ds(_GA, _GB)]], wrows.at[b, pl.ds(_GA, _GB)],
          gsem[b]).wait()

    def pos_fill(rl, b):
      @pl.loop(0, _L - 8, step=16)
      def _tok(i0):
        pvec = pi[rl, pl.ds(i0, 16)]
        for j in range(16):
          row = lax.broadcast(pvec[j], (16,))
          prows[b, i0 + j, :] = plsc.load_gather(wp_v, [row, lanes])

      pvec = pi[rl, pl.ds(_L - 16, 16)]
      for j in range(8, 16):
        row = lax.broadcast(pvec[j], (16,))
        prows[b, _L - 16 + j, :] = plsc.load_gather(wp_v, [row, lanes])

    def issue_write(row, b):
      pltpu.async_copy(
          wrows.at[b], out_hbm.at[row, pl.ds(0, _L), pl.ds(0, _DW)], wsem[b])
      pltpu.async_copy(
          prows.at[b], out_hbm.at[row, pl.ds(0, _L), pl.ds(_DW, _DP)],
          wsem[b])

    def wait_write(b):
      pltpu.make_async_copy(
          wrows.at[b], out_hbm.at[0, pl.ds(0, _L), pl.ds(0, _DW)],
          wsem[b]).wait()
      pltpu.make_async_copy(
          prows.at[b], out_hbm.at[0, pl.ds(0, _L), pl.ds(_DW, _DP)],
          wsem[b]).wait()

    @pl.loop(0, _NCH)
    def _chunk(c):
      crow = rowbase + c * _CR
      pltpu.sync_copy(x_hbm.at[pl.ds(crow, _CR)], xi)
      pltpu.sync_copy(p_hbm.at[pl.ds(crow, _CR)], pi)
      issue_gather(0, 0)
      issue_gather(1, 1)

      @pl.loop(0, _CR, step=2)
      def _rows(rl0):
        for b in range(2):
          rl = rl0 + b
          pos_fill(rl, b)
          wait_gather(b)
          issue_write(crow + rl, b)
          wait_write(b)

          @pl.when(rl + 2 < _CR)
          def _():
            issue_gather(rl + 2, b)

  return emb


_emb = _build()


@jax.jit
def kernel(x, pos, W_word, W_pos):
  return _emb(x.astype(jnp.int32), pos.astype(jnp.int32), W_word, W_pos)


# speedup vs baseline: 2.2866x; 1.0016x over previous
import functools

import jax
import jax.numpy as jnp
from jax import lax
from jax.experimental import pallas as pl
from jax.experimental.pallas import tpu as pltpu
from jax.experimental.pallas import tpu_sc as plsc

_B, _L = 4096, 200
_N = _B * _L
_DW, _DP = 128, 16
_D = _DW + _DP
_NC, _NS = 2, 16
_NW = _NC * _NS
_RW = _B // _NW
_CR = 32
_NCH = _RW // _CR
_GA, _GB = 128, _L - 128


def _build():
  mesh = plsc.VectorSubcoreMesh(core_axis_name="c", subcore_axis_name="s")

  @functools.partial(
      pl.kernel,
      mesh=mesh,
      compiler_params=pltpu.CompilerParams(
          needs_layout_passes=False, skip_device_barrier=True),
      out_type=jax.ShapeDtypeStruct((_B, _L, _D), jnp.float32),
      scratch_types=[
          pltpu.VMEM((_CR, _L), jnp.int32),
          pltpu.VMEM((_CR, _L), jnp.int32),
          pltpu.VMEM((24, _DP), jnp.float32),
          pltpu.VMEM((2, _L, _DW), jnp.float32),
          pltpu.VMEM((2, _L, _DP), jnp.float32),
          pltpu.SemaphoreType.DMA,
          pltpu.SemaphoreType.DMA,
          pltpu.SemaphoreType.DMA,
          pltpu.SemaphoreType.DMA,
      ],
  )
  def emb(x_hbm, p_hbm, ww_hbm, wp_hbm, out_hbm,
          xi, pi, wp_v, wrows, prows, gs0, gs1, ws0, ws1):
    gsem = (gs0, gs1)
    wsem = (ws0, ws1)
    wid = lax.axis_index("s") * _NC + lax.axis_index("c")
    rowbase = wid * _RW
    lanes = lax.iota(jnp.int32, 16)
    pltpu.sync_copy(wp_hbm, wp_v)

    def issue_gather(rl, b):
      pltpu.async_copy(ww_hbm.at[xi.at[rl, pl.ds(0, _GA)]],
                       wrows.at[b, pl.ds(0, _GA)], gsem[b])
      pltpu.async_copy(ww_hbm.at[xi.at[rl, pl.ds(_GA, _GB)]],
                       wrows.at[b, pl.ds(_GA, _GB)], gsem[b])

    def wait_gather(b):
      pltpu.make_async_copy(
          ww_hbm.at[xi.at[0, pl.ds(0, _GA)]], wrows.at[b, pl.ds(0, _GA)],
          gsem[b]).wait()
      pltpu.make_async_copy(
          ww_hbm.at[xi.at[0, pl.ds(_GA, _GB)]], wrows.at[b, pl.ds(_GA, _GB)],
          gsem[b]).wait()

    def pos_fill(rl, b):
      @pl.loop(0, _L - 8, step=16)
      def _tok(i0):
        pvec = pi[rl, pl.ds(i0, 16)]
        for j in range(16):
          row = lax.broadcast(pvec[j], (16,))
          prows[b, i0 + j, :] = plsc.load_gather(wp_v, [row, lanes])

      pvec = pi[rl, pl.ds(_L - 16, 16)]
      for j in range(8, 16):
        row = lax.broadcast(pvec[j], (16,))
        prows[b, _L - 16 + j, :] = plsc.load_gather(wp_v, [row, lanes])

    def issue_write(row, b):
      pltpu.async_copy(
          wrows.at[b], out_hbm.at[row, pl.ds(0, _L), pl.ds(0, _DW)], wsem[b])
      pltpu.async_copy(
          prows.at[b], out_hbm.at[row, pl.ds(0, _L), pl.ds(_DW, _DP)],
          wsem[b])

    def wait_write(b):
      pltpu.make_async_copy(
          wrows.at[b], out_hbm.at[0, pl.ds(0, _L), pl.ds(0, _DW)],
          wsem[b]).wait()
      pltpu.make_async_copy(
          prows.at[b], out_hbm.at[0, pl.ds(0, _L), pl.ds(_DW, _DP)],
          wsem[b]).wait()

    @pl.loop(0, _NCH)
    def _chunk(c):
      crow = rowbase + c * _CR
      pltpu.sync_copy(x_hbm.at[pl.ds(crow, _CR)], xi)
      pltpu.sync_copy(p_hbm.at[pl.ds(crow, _CR)], pi)
      issue_gather(0, 0)
      issue_gather(1, 1)

      @pl.loop(0, _CR, step=2)
      def _rows(rl0):
        for b in range(2):
          rl = rl0 + b
          pos_fill(rl, b)
          wait_gather(b)
          issue_write(crow + rl, b)
          wait_write(b)

          @pl.when(rl + 2 < _CR)
          def _():
            issue_gather(rl + 2, b)

  return emb


_emb = _build()


@jax.jit
def kernel(x, pos, W_word, W_pos):
  return _emb(x.astype(jnp.int32), pos.astype(jnp.int32), W_word, W_pos)
